# baseline (speedup 1.0000x reference)
import functools

import jax
import jax.numpy as jnp
from jax import lax
from jax.experimental import pallas as pl
from jax.experimental.pallas import tpu as pltpu
from jax.experimental.pallas import tpu_sc as plsc

NC = 2
NS = 16
NW = NC * NS
CHUNK = 128
LANES = 16


def _sc_update(B, D, nchunks):
    mesh = plsc.VectorSubcoreMesh(core_axis_name="c", subcore_axis_name="s")

    @functools.partial(
        pl.kernel,
        out_type=(),
        mesh=mesh,
        compiler_params=pltpu.CompilerParams(
            needs_layout_passes=False, use_tc_tiling_on_sc=False),
        scratch_types=[
            pltpu.VMEM((CHUNK,), jnp.int32),
            pltpu.VMEM((CHUNK,), jnp.int32),
            pltpu.VMEM((CHUNK, D), jnp.float32),
            pltpu.VMEM((CHUNK, D), jnp.float32),
            pltpu.VMEM((LANES,), jnp.float32),
            pltpu.SemaphoreType.DMA,
        ],
    )
    def body(idx_hbm, pos_hbm, mem_hbm, mom_hbm, memory_hbm, table,
             idxv, posv, oldv, newv, momv, sem):
        c = lax.axis_index("c")
        s = lax.axis_index("s")
        w = s * NC + c
        pltpu.sync_copy(mom_hbm, momv)
        mval = momv[...]
        one_m = 1.0 - mval

        @pl.loop(0, nchunks)
        def _chunk(j):
            pltpu.sync_copy(idx_hbm.at[w, j], idxv)
            pltpu.sync_copy(pos_hbm.at[w, j], posv)
            pltpu.async_copy(memory_hbm.at[idxv], oldv, sem).wait()
            pltpu.async_copy(mem_hbm.at[posv], newv, sem).wait()

            @pl.loop(0, CHUNK)
            def _row(r):
                acc = jnp.zeros((LANES,), jnp.float32)
                for k in range(D // LANES):
                    o = oldv[r, pl.ds(k * LANES, LANES)]
                    n = newv[r, pl.ds(k * LANES, LANES)]
                    u = o * one_m + n * mval
                    oldv[r, pl.ds(k * LANES, LANES)] = u
                    acc = acc + u * u
                ssum = jnp.sum(acc)
                sv = lax.broadcast_in_dim(ssum, (LANES,), ())
                iv = plsc.bitcast(sv, jnp.int32)
                iv = jnp.int32(0x5F3759DF) - lax.shift_right_logical(iv, 1)
                y = plsc.bitcast(iv, jnp.float32)
                for _ in range(3):
                    y = y * (1.5 - 0.5 * sv * y * y)
                for k in range(D // LANES):
                    oldv[r, pl.ds(k * LANES, LANES)] = (
                        oldv[r, pl.ds(k * LANES, LANES)] * y)

            pltpu.async_copy(oldv, table.at[idxv], sem).wait()

    return body


def kernel(mem, momentum, ind, time, memory):
    mem2 = mem.reshape(mem.shape[0], -1)
    B, D = mem2.shape
    ind32 = ind.astype(jnp.int32)
    nchunks = B // (NW * CHUNK)

    perm = jnp.argsort(ind32, stable=True)
    ind_s = ind32[perm]
    iota = jnp.arange(B, dtype=jnp.int32)
    is_end = jnp.concatenate(
        [ind_s[1:] != ind_s[:-1], jnp.ones((1,), bool)])
    run_end = jnp.flip(lax.cummin(jnp.flip(
        jnp.where(is_end, iota, B - 1).astype(jnp.int32))))
    winner_pos = perm[run_end]

    idx3 = ind_s.reshape(NW, nchunks, CHUNK)
    pos3 = winner_pos.reshape(NW, nchunks, CHUNK)
    mom16 = jnp.full((LANES,), momentum, jnp.float32)

    table_ref = jax.new_ref(memory)
    _sc_update(B, D, nchunks)(idx3, pos3, mem2, mom16, memory, table_ref)
    return table_ref[...]


# speedup vs baseline: 1.0044x; 1.0044x over previous
import functools

import jax
import jax.numpy as jnp
from jax import lax
from jax.experimental import pallas as pl
from jax.experimental.pallas import tpu as pltpu
from jax.experimental.pallas import tpu_sc as plsc

NC = 2
NS = 16
NW = NC * NS
CHUNK = 128
LANES = 16


def _sc_update(B, D, nchunks):
    mesh = plsc.VectorSubcoreMesh(core_axis_name="c", subcore_axis_name="s")

    @functools.partial(
        pl.kernel,
        out_type=(),
        mesh=mesh,
        compiler_params=pltpu.CompilerParams(
            needs_layout_passes=False, use_tc_tiling_on_sc=False),
        scratch_types=[
            pltpu.VMEM((CHUNK,), jnp.int32),
            pltpu.VMEM((CHUNK,), jnp.int32),
            pltpu.VMEM((CHUNK, D), jnp.float32),
            pltpu.VMEM((CHUNK, D), jnp.float32),
            pltpu.VMEM((LANES,), jnp.float32),
            pltpu.SemaphoreType.DMA,
        ],
    )
    def body(idx_hbm, pos_hbm, mem_hbm, mom_hbm, memory_hbm, table,
             idxv, posv, oldv, newv, momv, sem):
        c = lax.axis_index("c")
        s = lax.axis_index("s")
        w = s * NC + c
        pltpu.sync_copy(mom_hbm, momv)
        mval = momv[...]
        one_m = 1.0 - mval

        @pl.loop(0, nchunks)
        def _chunk(j):
            pltpu.sync_copy(idx_hbm.at[w, j], idxv)
            pltpu.sync_copy(pos_hbm.at[w, j], posv)
            pltpu.async_copy(memory_hbm.at[idxv], oldv, sem).wait()
            pltpu.async_copy(mem_hbm.at[posv], newv, sem).wait()

            @pl.loop(0, CHUNK)
            def _row(r):
                acc = jnp.zeros((LANES,), jnp.float32)
                for k in range(D // LANES):
                    o = oldv[r, pl.ds(k * LANES, LANES)]
                    n = newv[r, pl.ds(k * LANES, LANES)]
                    u = o * one_m + n * mval
                    oldv[r, pl.ds(k * LANES, LANES)] = u
                    acc = acc + u * u
                ssum = jnp.sum(acc)
                sv = lax.broadcast_in_dim(ssum, (LANES,), ())
                iv = plsc.bitcast(sv, jnp.int32)
                iv = jnp.int32(0x5F3759DF) - lax.shift_right_logical(iv, 1)
                y = plsc.bitcast(iv, jnp.float32)
                for _ in range(3):
                    y = y * (1.5 - 0.5 * sv * y * y)
                for k in range(D // LANES):
                    oldv[r, pl.ds(k * LANES, LANES)] = (
                        oldv[r, pl.ds(k * LANES, LANES)] * y)

            pltpu.async_copy(oldv, table.at[idxv], sem).wait()

    return body


def kernel(mem, momentum, ind, time, memory):
    mem2 = mem.reshape(mem.shape[0], -1)
    B, D = mem2.shape
    ind32 = ind.astype(jnp.int32)
    nchunks = B // (NW * CHUNK)

    iota = jnp.arange(B, dtype=jnp.int32)
    pos_table = jnp.zeros((memory.shape[0],), jnp.int32).at[ind32].set(iota)
    winner_pos = pos_table[ind32]

    idx3 = ind32.reshape(NW, nchunks, CHUNK)
    pos3 = winner_pos.reshape(NW, nchunks, CHUNK)
    mom16 = jnp.full((LANES,), momentum, jnp.float32)

    table_ref = jax.new_ref(memory)
    _sc_update(B, D, nchunks)(idx3, pos3, mem2, mom16, memory, table_ref)
    return jax.freeze(table_ref)
